# Initial kernel scaffold; baseline (speedup 1.0000x reference)
import jax
import jax.numpy as jnp
from jax.experimental import pallas as pl
from jax.experimental.pallas import tpu as pltpu

E = 8
H = 1024
I_MOE = 512
I_SHARED = 1024


def _dot_t(a, b):
    return jax.lax.dot_general(
        a, b, (((1,), (1,)), ((), ())), preferred_element_type=jnp.float32
    )


def _silu(x):
    return x * jax.nn.sigmoid(x)


def _moe_kernel(x_ref, gate_w_ref, gp_ref, up_ref, dp_ref,
                sg_ref, su_ref, sd_ref, seg_ref,
                out_ref, combine_ref):
    e = pl.program_id(0)
    x = x_ref[...]
    t = x.shape[0]

    @pl.when(e == 0)
    def _router_and_shared():
        logits = _dot_t(x, gate_w_ref[...])
        m = jnp.max(logits, axis=-1, keepdims=True)
        p = jnp.exp(logits - m)
        p = p / jnp.sum(p, axis=-1, keepdims=True)

        e_iota = jax.lax.broadcasted_iota(jnp.int32, (t, E), 1)
        w1 = jnp.max(p, axis=-1, keepdims=True)
        i1 = jnp.min(jnp.where(p == w1, e_iota, E), axis=-1, keepdims=True)
        m1 = e_iota == i1
        p2 = jnp.where(m1, -1.0, p)
        w2 = jnp.max(p2, axis=-1, keepdims=True)
        i2 = jnp.min(jnp.where(p2 == w2, e_iota, E), axis=-1, keepdims=True)
        m2 = e_iota == i2
        denom = w1 + w2
        combine = jnp.where(m1, w1, 0.0) + jnp.where(m2, w2, 0.0)
        combine_ref[...] = combine / denom

        sg = _dot_t(x, sg_ref[...])
        su = _dot_t(x, su_ref[...])
        shared = _dot_t(_silu(sg) * su, sd_ref[...])
        gate_val = jax.nn.sigmoid(_dot_t(x, seg_ref[...]))
        out_ref[...] = gate_val * shared

    g = _dot_t(x, gp_ref[0])
    u = _dot_t(x, up_ref[0])
    act = _silu(g) * u
    w_e = combine_ref[:, pl.ds(e, 1)]
    out_ref[...] += _dot_t(act * w_e, dp_ref[0])


def kernel(hidden_states, gate_w, gate_proj_w, up_proj_w, down_proj_w,
           shared_gate_w, shared_up_w, shared_down_w, shared_expert_gate_w):
    b, s, h = hidden_states.shape
    x = hidden_states.reshape(-1, h)
    t = x.shape[0]

    out = pl.pallas_call(
        _moe_kernel,
        grid=(E,),
        in_specs=[
            pl.BlockSpec((t, h), lambda e: (0, 0)),
            pl.BlockSpec((E, h), lambda e: (0, 0)),
            pl.BlockSpec((1, I_MOE, h), lambda e: (e, 0, 0)),
            pl.BlockSpec((1, I_MOE, h), lambda e: (e, 0, 0)),
            pl.BlockSpec((1, h, I_MOE), lambda e: (e, 0, 0)),
            pl.BlockSpec((I_SHARED, h), lambda e: (0, 0)),
            pl.BlockSpec((I_SHARED, h), lambda e: (0, 0)),
            pl.BlockSpec((h, I_SHARED), lambda e: (0, 0)),
            pl.BlockSpec((1, h), lambda e: (0, 0)),
        ],
        out_specs=pl.BlockSpec((t, h), lambda e: (0, 0)),
        out_shape=jax.ShapeDtypeStruct((t, h), jnp.float32),
        scratch_shapes=[pltpu.VMEM((t, E), jnp.float32)],
    )(x, gate_w, gate_proj_w, up_proj_w, down_proj_w,
      shared_gate_w, shared_up_w, shared_down_w, shared_expert_gate_w)

    return out.reshape(b, s, h)


# speedup vs baseline: 1.9716x; 1.9716x over previous
import jax
import jax.numpy as jnp
from jax.experimental import pallas as pl
from jax.experimental.pallas import tpu as pltpu

E = 8
H = 1024
I_MOE = 512
I_SHARED = 1024


def _dot_t(a, b):
    return jax.lax.dot_general(
        a, b, (((1,), (1,)), ((), ())), preferred_element_type=jnp.float32
    )


def _silu(x):
    return x * jax.nn.sigmoid(x)


def _moe_kernel(x_ref, gate_w_ref, gp_ref, up_ref, dp_ref,
                sg_ref, su_ref, sd_ref, seg_ref,
                out_ref, combine_ref):
    e = pl.program_id(0)
    x = x_ref[...]
    t = x.shape[0]

    @pl.when(e == 0)
    def _router_and_shared():
        logits = _dot_t(x, gate_w_ref[...])
        m = jnp.max(logits, axis=-1, keepdims=True)
        p = jnp.exp(logits - m)
        p = p / jnp.sum(p, axis=-1, keepdims=True)

        e_iota = jax.lax.broadcasted_iota(jnp.int32, (t, E), 1)
        w1 = jnp.max(p, axis=-1, keepdims=True)
        i1 = jnp.min(jnp.where(p == w1, e_iota, E), axis=-1, keepdims=True)
        m1 = e_iota == i1
        p2 = jnp.where(m1, -1.0, p)
        w2 = jnp.max(p2, axis=-1, keepdims=True)
        i2 = jnp.min(jnp.where(p2 == w2, e_iota, E), axis=-1, keepdims=True)
        m2 = e_iota == i2
        denom = w1 + w2
        combine = jnp.where(m1, w1, 0.0) + jnp.where(m2, w2, 0.0)
        combine_ref[...] = combine / denom

        sg = _dot_t(x, sg_ref[...])
        su = _dot_t(x, su_ref[...])
        shared = _dot_t(_silu(sg) * su, sd_ref[...])
        gate_val = jax.nn.sigmoid(_dot_t(x, seg_ref[...]))
        out_ref[...] = gate_val * shared

    g = _dot_t(x, gp_ref[0])
    u = _dot_t(x, up_ref[0])
    act = _silu(g) * u
    combine = combine_ref[...]
    col = jax.lax.broadcasted_iota(jnp.int32, (t, E), 1) == e
    w_e = jnp.sum(jnp.where(col, combine, 0.0), axis=-1, keepdims=True)
    out_ref[...] += _dot_t(act * w_e, dp_ref[0])


def kernel(hidden_states, gate_w, gate_proj_w, up_proj_w, down_proj_w,
           shared_gate_w, shared_up_w, shared_down_w, shared_expert_gate_w):
    b, s, h = hidden_states.shape
    x = hidden_states.reshape(-1, h)
    t = x.shape[0]

    out = pl.pallas_call(
        _moe_kernel,
        grid=(E,),
        in_specs=[
            pl.BlockSpec((t, h), lambda e: (0, 0)),
            pl.BlockSpec((E, h), lambda e: (0, 0)),
            pl.BlockSpec((1, I_MOE, h), lambda e: (e, 0, 0)),
            pl.BlockSpec((1, I_MOE, h), lambda e: (e, 0, 0)),
            pl.BlockSpec((1, h, I_MOE), lambda e: (e, 0, 0)),
            pl.BlockSpec((I_SHARED, h), lambda e: (0, 0)),
            pl.BlockSpec((I_SHARED, h), lambda e: (0, 0)),
            pl.BlockSpec((h, I_SHARED), lambda e: (0, 0)),
            pl.BlockSpec((1, h), lambda e: (0, 0)),
        ],
        out_specs=pl.BlockSpec((t, h), lambda e: (0, 0)),
        out_shape=jax.ShapeDtypeStruct((t, h), jnp.float32),
        scratch_shapes=[pltpu.VMEM((t, E), jnp.float32)],
    )(x, gate_w, gate_proj_w, up_proj_w, down_proj_w,
      shared_gate_w, shared_up_w, shared_down_w, shared_expert_gate_w)

    return out.reshape(b, s, h)
